# baseline (speedup 1.0000x reference)
import functools

import jax
import jax.numpy as jnp
from jax.experimental import pallas as pl
from jax.experimental.pallas import tpu as pltpu

_LANE = 128

_SELU_ALPHA = 1.6732632423543772
_SELU_SCALE = 1.0507009873554805
_LOG2E = 1.4426950408889634
_LN2 = 0.6931471805599453
_ALPHA2 = _SELU_ALPHA / _LN2

_E1_DIMS = [(16, 3), (32, 16), (64, 32), (256, 64),
            (64, 256), (32, 64), (16, 32), (1, 16)]
_E2_DIMS = [(256, 64), (64, 32), (32, 16), (16, 1)]


def _round_up(v, m):
    return ((v + m - 1) // m) * m


def _col_offsets(widths):
    offs, c = [], 0
    for w in widths:
        offs.append(c)
        c += _round_up(w, _LANE)
    return offs, c


_E1_COLS, _E1_TOTAL = _col_offsets([fi for _, fi in _E1_DIMS])
_E2_COLS, _E2_TOTAL = _col_offsets([fo for _, fo in _E2_DIMS])

_T_DIMS = [(16, 3), (32, 16), (256, 32), (64, 256), (32, 64), (16, 32), (1, 16)]
_T_COLS, _T_TOTAL = _col_offsets([fi for _, fi in _T_DIMS])
_T_SELU = (True, True, True, True, True, True, False)


def _prep_params(w1, b1, w2, b2):
    ws, bs = [], []
    for l, ((fo, fi), col) in enumerate(zip(_E1_DIMS, _E1_COLS)):
        ws.append(w1[0:fo, col:col + fi])
        bs.append(b1[0:fo, l])
    w34 = ws[3] @ ws[2]
    b34 = ws[3] @ bs[2] + bs[3]
    tw = [ws[0], ws[1], w34, ws[4], ws[5], ws[6], ws[7]]
    tb = [bs[0], bs[1], b34, bs[4], bs[5], bs[6], bs[7]]
    for l in range(1, len(tw)):
        if _T_SELU[l - 1]:
            tw[l] = tw[l] * _SELU_SCALE
    for l in range(len(tw)):
        tw[l] = tw[l] * (_LOG2E * (_LN2 if l > 0 else 1.0))
        tb[l] = tb[l] * _LOG2E

    w1p = jnp.zeros((256, _T_TOTAL), jnp.float32)
    b1p = jnp.zeros((256, len(_T_DIMS)), jnp.float32)
    for l, ((fo, fi), col) in enumerate(zip(_T_DIMS, _T_COLS)):
        w1p = w1p.at[0:fo, col:col + fi].set(tw[l])
        b1p = b1p.at[0:fo, l].set(tb[l])

    w2p = w2
    for m in range(1, len(_E2_DIMS)):
        fi, fo = _E2_DIMS[m]
        col = _E2_COLS[m]
        w2p = w2p.at[0:fi, col:col + fo].multiply(_SELU_SCALE)
    return w1p.astype(jnp.float8_e4m3fn), b1p, w2p, b2


def _selu_noscale(x):
    e = jnp.exp(x)
    a = jnp.asarray(_SELU_ALPHA, x.dtype)
    return jnp.minimum(x, a * e - a)


def _selu_exp2(u):
    e = jnp.exp2(u)
    a = jnp.asarray(_ALPHA2, u.dtype)
    return jnp.minimum(u, a * e - a)


def _sigmoid_exp2(u):
    return 1.0 / (1.0 + jnp.exp2(-u))


def _sigmoid(x):
    return 1.0 / (1.0 + jnp.exp(-x))


def _trunk(xt_ref, w1_ref, b1_ref, c, s):
    h = xt_ref[:, c * s:(c + 1) * s]
    for l, ((fo, fi), col) in enumerate(zip(_T_DIMS, _T_COLS)):
        w = w1_ref[0:fo, col:col + fi]
        b = b1_ref[0:fo, l:l + 1]
        z = jnp.dot(w, h, preferred_element_type=jnp.float32) + b
        if _T_SELU[l]:
            h = _selu_exp2(z.astype(jnp.bfloat16)).astype(jnp.float8_e4m3fn)
        else:
            h = _sigmoid_exp2(z)
    return h.astype(jnp.bfloat16)


def _fused_kernel(xt_ref, pool_ref, w1_ref, b1_ref, w2_ref, b2_ref, out_ref,
                  *, bb, n, chunks):
    s = (bb // chunks) * n
    rpc = bb // chunks
    pool = pool_ref[...]
    parts = []
    for c in range(chunks):
        hb = _trunk(xt_ref, w1_ref, b1_ref, c, s)
        parts.append(hb.reshape(rpc, n))
    hrows = jnp.concatenate(parts, axis=0)
    z = jnp.dot(hrows, pool, preferred_element_type=jnp.float32)

    for m, ((fi, fo), col) in enumerate(zip(_E2_DIMS, _E2_COLS)):
        w = w2_ref[0:fi, col:col + fo]
        b = b2_ref[m:m + 1, 0:fo]
        z = jnp.dot(z, w, preferred_element_type=jnp.float32) + b
        z = _selu_noscale(z) if m + 1 < len(_E2_DIMS) else _sigmoid(z)
    out_ref[...] = z.reshape(bb, 1, 1)


def kernel(x, w1, b1, w2, b2, pool_mat):
    B, N, F = x.shape
    assert F == 3
    BB = 64
    assert B % BB == 0
    S = BB * N

    xt = jnp.transpose(x, (2, 0, 1)).reshape(3, B * N).astype(jnp.float8_e4m3fn)
    w1p, b1p, w2p, b2p = _prep_params(w1, b1, w2, b2)

    body = functools.partial(_fused_kernel, bb=BB, n=N, chunks=8)
    poolb = pool_mat.astype(jnp.bfloat16)
    return pl.pallas_call(
        body,
        out_shape=jax.ShapeDtypeStruct((B, 1, 1), jnp.float32),
        grid=(B // BB,),
        in_specs=[
            pl.BlockSpec((3, S), lambda i: (0, i)),
            pl.BlockSpec(poolb.shape, lambda i: (0, 0)),
            pl.BlockSpec(w1p.shape, lambda i: (0, 0)),
            pl.BlockSpec(b1p.shape, lambda i: (0, 0)),
            pl.BlockSpec(w2p.shape, lambda i: (0, 0)),
            pl.BlockSpec(b2p.shape, lambda i: (0, 0)),
        ],
        out_specs=pl.BlockSpec((BB, 1, 1), lambda i: (i, 0, 0)),
        compiler_params=pltpu.CompilerParams(
            dimension_semantics=("parallel",),
            vmem_limit_bytes=64 * 1024 * 1024),
    )(xt, poolb, w1p, b1p, w2p, b2p)


# speedup vs baseline: 1.1802x; 1.1802x over previous
import functools

import jax
import jax.numpy as jnp
from jax.experimental import pallas as pl
from jax.experimental.pallas import tpu as pltpu

_LANE = 128

_SELU_ALPHA = 1.6732632423543772
_SELU_SCALE = 1.0507009873554805
_LOG2E = 1.4426950408889634
_LN2 = 0.6931471805599453
_ALPHA2 = _SELU_ALPHA / _LN2

_E1_DIMS = [(16, 3), (32, 16), (64, 32), (256, 64),
            (64, 256), (32, 64), (16, 32), (1, 16)]
_E2_DIMS = [(256, 64), (64, 32), (32, 16), (16, 1)]


def _round_up(v, m):
    return ((v + m - 1) // m) * m


def _col_offsets(widths):
    offs, c = [], 0
    for w in widths:
        offs.append(c)
        c += _round_up(w, _LANE)
    return offs, c


_E1_COLS, _E1_TOTAL = _col_offsets([fi for _, fi in _E1_DIMS])
_E2_COLS, _E2_TOTAL = _col_offsets([fo for _, fo in _E2_DIMS])

_T_DIMS = [(16, 3), (32, 16), (256, 32), (64, 256), (32, 64), (16, 32), (1, 16)]
_T_COLS, _T_TOTAL = _col_offsets([fi for _, fi in _T_DIMS])
_T_SELU = (True, True, True, True, True, True, False)


def _prep_params(w1, b1, w2, b2):
    ws, bs = [], []
    for l, ((fo, fi), col) in enumerate(zip(_E1_DIMS, _E1_COLS)):
        ws.append(w1[0:fo, col:col + fi])
        bs.append(b1[0:fo, l])
    w34 = ws[3] @ ws[2]
    b34 = ws[3] @ bs[2] + bs[3]
    tw = [ws[0], ws[1], w34, ws[4], ws[5], ws[6], ws[7]]
    tb = [bs[0], bs[1], b34, bs[4], bs[5], bs[6], bs[7]]
    for l in range(1, len(tw)):
        if _T_SELU[l - 1]:
            tw[l] = tw[l] * _SELU_SCALE
    for l in range(len(tw)):
        tw[l] = tw[l] * (_LOG2E * (_LN2 if l > 0 else 1.0))
        tb[l] = tb[l] * _LOG2E

    w1p = jnp.zeros((256, _T_TOTAL), jnp.float32)
    b1p = jnp.zeros((256, len(_T_DIMS)), jnp.float32)
    for l, ((fo, fi), col) in enumerate(zip(_T_DIMS, _T_COLS)):
        w1p = w1p.at[0:fo, col:col + fi].set(tw[l])
        b1p = b1p.at[0:fo, l].set(tb[l])

    w2p = w2
    for m in range(1, len(_E2_DIMS)):
        fi, fo = _E2_DIMS[m]
        col = _E2_COLS[m]
        w2p = w2p.at[0:fi, col:col + fo].multiply(_SELU_SCALE)
    return w1p.astype(jnp.float8_e4m3fn), b1p, w2p, b2


def _selu_noscale(x):
    e = jnp.exp(x)
    a = jnp.asarray(_SELU_ALPHA, x.dtype)
    return jnp.minimum(x, a * e - a)


def _selu_exp2(u):
    e = jnp.exp2(u)
    a = jnp.asarray(_ALPHA2, u.dtype)
    return jnp.minimum(u, a * e - a)


def _sigmoid_exp2(u):
    return 1.0 / (1.0 + jnp.exp2(-u))


def _sigmoid(x):
    return 1.0 / (1.0 + jnp.exp(-x))


def _trunk(xt_ref, w1_ref, b1_ref, c, s):
    h = xt_ref[:, c * s:(c + 1) * s]
    for l, ((fo, fi), col) in enumerate(zip(_T_DIMS, _T_COLS)):
        w = w1_ref[0:fo, col:col + fi]
        b = b1_ref[0:fo, l:l + 1]
        z = jnp.dot(w, h, preferred_element_type=jnp.float32) + b
        if _T_SELU[l]:
            h = _selu_exp2(z.astype(jnp.bfloat16)).astype(jnp.float8_e4m3fn)
        else:
            h = _sigmoid_exp2(z)
    return h.astype(jnp.bfloat16)


def _fused_kernel(xt_ref, pool_ref, w1_ref, b1_ref, w2_ref, b2_ref, out_ref,
                  *, bb, n, chunks):
    s = (bb // chunks) * n
    rpc = bb // chunks
    pool = pool_ref[...]
    parts = []
    for c in range(chunks):
        hb = _trunk(xt_ref, w1_ref, b1_ref, c, s)
        parts.append(hb.reshape(rpc, n))
    hrows = jnp.concatenate(parts, axis=0)
    z = jnp.dot(hrows, pool, preferred_element_type=jnp.float32)

    for m, ((fi, fo), col) in enumerate(zip(_E2_DIMS, _E2_COLS)):
        w = w2_ref[0:fi, col:col + fo]
        b = b2_ref[m:m + 1, 0:fo]
        z = jnp.dot(z, w, preferred_element_type=jnp.float32) + b
        z = _selu_noscale(z) if m + 1 < len(_E2_DIMS) else _sigmoid(z)
    out_ref[...] = z.reshape(bb, 1, 1)


def kernel(x, w1, b1, w2, b2, pool_mat):
    B, N, F = x.shape
    assert F == 3
    BB = 32
    assert B % BB == 0
    S = BB * N

    xt = jnp.transpose(x, (2, 0, 1)).reshape(3, B * N).astype(jnp.float8_e4m3fn)
    w1p, b1p, w2p, b2p = _prep_params(w1, b1, w2, b2)

    body = functools.partial(_fused_kernel, bb=BB, n=N, chunks=2)
    poolb = pool_mat.astype(jnp.bfloat16)
    return pl.pallas_call(
        body,
        out_shape=jax.ShapeDtypeStruct((B, 1, 1), jnp.float32),
        grid=(B // BB,),
        in_specs=[
            pl.BlockSpec((3, S), lambda i: (0, i)),
            pl.BlockSpec(poolb.shape, lambda i: (0, 0)),
            pl.BlockSpec(w1p.shape, lambda i: (0, 0)),
            pl.BlockSpec(b1p.shape, lambda i: (0, 0)),
            pl.BlockSpec(w2p.shape, lambda i: (0, 0)),
            pl.BlockSpec(b2p.shape, lambda i: (0, 0)),
        ],
        out_specs=pl.BlockSpec((BB, 1, 1), lambda i: (i, 0, 0)),
        compiler_params=pltpu.CompilerParams(
            dimension_semantics=("parallel",),
            vmem_limit_bytes=64 * 1024 * 1024),
    )(xt, poolb, w1p, b1p, w2p, b2p)


# speedup vs baseline: 1.2238x; 1.0369x over previous
import functools

import jax
import jax.numpy as jnp
from jax.experimental import pallas as pl
from jax.experimental.pallas import tpu as pltpu

_LANE = 128

_SELU_ALPHA = 1.6732632423543772
_SELU_SCALE = 1.0507009873554805
_LOG2E = 1.4426950408889634
_LN2 = 0.6931471805599453
_ALPHA2 = _SELU_ALPHA / _LN2

_E1_DIMS = [(16, 3), (32, 16), (64, 32), (256, 64),
            (64, 256), (32, 64), (16, 32), (1, 16)]
_E2_DIMS = [(256, 64), (64, 32), (32, 16), (16, 1)]


def _round_up(v, m):
    return ((v + m - 1) // m) * m


def _col_offsets(widths):
    offs, c = [], 0
    for w in widths:
        offs.append(c)
        c += _round_up(w, _LANE)
    return offs, c


_E1_COLS, _E1_TOTAL = _col_offsets([fi for _, fi in _E1_DIMS])
_E2_COLS, _E2_TOTAL = _col_offsets([fo for _, fo in _E2_DIMS])

_T_DIMS = [(16, 3), (32, 16), (256, 32), (64, 256), (32, 64), (16, 32), (1, 16)]
_T_COLS, _T_TOTAL = _col_offsets([fi for _, fi in _T_DIMS])
_T_SELU = (True, True, True, True, True, True, False)


def _prep_params(w1, b1, w2, b2):
    ws, bs = [], []
    for l, ((fo, fi), col) in enumerate(zip(_E1_DIMS, _E1_COLS)):
        ws.append(w1[0:fo, col:col + fi])
        bs.append(b1[0:fo, l])
    w34 = ws[3] @ ws[2]
    b34 = ws[3] @ bs[2] + bs[3]
    tw = [ws[0], ws[1], w34, ws[4], ws[5], ws[6], ws[7]]
    tb = [bs[0], bs[1], b34, bs[4], bs[5], bs[6], bs[7]]
    for l in range(1, len(tw)):
        if _T_SELU[l - 1]:
            tw[l] = tw[l] * _SELU_SCALE
    for l in range(len(tw)):
        tw[l] = tw[l] * (_LOG2E * (_LN2 if l > 0 else 1.0))
        tb[l] = tb[l] * _LOG2E

    w1p = jnp.zeros((256, _T_TOTAL), jnp.float32)
    b1p = jnp.zeros((256, len(_T_DIMS)), jnp.float32)
    for l, ((fo, fi), col) in enumerate(zip(_T_DIMS, _T_COLS)):
        w1p = w1p.at[0:fo, col:col + fi].set(tw[l])
        b1p = b1p.at[0:fo, l].set(tb[l])

    w2p = w2
    for m in range(1, len(_E2_DIMS)):
        fi, fo = _E2_DIMS[m]
        col = _E2_COLS[m]
        w2p = w2p.at[0:fi, col:col + fo].multiply(_SELU_SCALE)
    return w1p.astype(jnp.float8_e4m3fn), b1p, w2p, b2


def _selu_noscale(x):
    e = jnp.exp(x)
    a = jnp.asarray(_SELU_ALPHA, x.dtype)
    return jnp.minimum(x, a * e - a)


def _selu_exp2(u):
    e = jnp.exp2(u)
    a = jnp.asarray(_ALPHA2, u.dtype)
    return jnp.minimum(u, a * e - a)


def _sigmoid_exp2(u):
    return 1.0 / (1.0 + jnp.exp2(-u))


def _sigmoid(x):
    return 1.0 / (1.0 + jnp.exp(-x))


def _trunk(xt_ref, w1_ref, b1_ref, c, s):
    h = xt_ref[:, c * s:(c + 1) * s]
    for l, ((fo, fi), col) in enumerate(zip(_T_DIMS, _T_COLS)):
        w = w1_ref[0:fo, col:col + fi]
        b = b1_ref[0:fo, l:l + 1]
        z = jnp.dot(w, h, preferred_element_type=jnp.float32) + b
        if _T_SELU[l]:
            h = _selu_exp2(z.astype(jnp.bfloat16)).astype(jnp.float8_e4m3fn)
        else:
            h = _sigmoid_exp2(z)
    return h.astype(jnp.bfloat16)


def _fused_kernel(xt_ref, pool_ref, w1_ref, b1_ref, w2_ref, b2_ref, out_ref,
                  *, bb, n, chunks):
    s = (bb // chunks) * n
    rpc = bb // chunks
    pool = pool_ref[...]
    parts = []
    for c in range(chunks):
        hb = _trunk(xt_ref, w1_ref, b1_ref, c, s)
        parts.append(hb.reshape(rpc, n))
    hrows = jnp.concatenate(parts, axis=0)
    z = jnp.dot(hrows, pool, preferred_element_type=jnp.float32)

    for m, ((fi, fo), col) in enumerate(zip(_E2_DIMS, _E2_COLS)):
        w = w2_ref[0:fi, col:col + fo]
        b = b2_ref[m:m + 1, 0:fo]
        z = jnp.dot(z, w, preferred_element_type=jnp.float32) + b
        z = _selu_noscale(z) if m + 1 < len(_E2_DIMS) else _sigmoid(z)
    out_ref[...] = z.reshape(bb, 1, 1)


def kernel(x, w1, b1, w2, b2, pool_mat):
    B, N, F = x.shape
    assert F == 3
    BB = 32
    assert B % BB == 0
    S = BB * N

    xt = jnp.transpose(x, (2, 0, 1)).reshape(3, B * N).astype(jnp.float8_e4m3fn)
    w1p, b1p, w2p, b2p = _prep_params(w1, b1, w2, b2)

    body = functools.partial(_fused_kernel, bb=BB, n=N, chunks=4)
    poolb = pool_mat.astype(jnp.bfloat16)
    return pl.pallas_call(
        body,
        out_shape=jax.ShapeDtypeStruct((B, 1, 1), jnp.float32),
        grid=(B // BB,),
        in_specs=[
            pl.BlockSpec((3, S), lambda i: (0, i)),
            pl.BlockSpec(poolb.shape, lambda i: (0, 0)),
            pl.BlockSpec(w1p.shape, lambda i: (0, 0)),
            pl.BlockSpec(b1p.shape, lambda i: (0, 0)),
            pl.BlockSpec(w2p.shape, lambda i: (0, 0)),
            pl.BlockSpec(b2p.shape, lambda i: (0, 0)),
        ],
        out_specs=pl.BlockSpec((BB, 1, 1), lambda i: (i, 0, 0)),
        compiler_params=pltpu.CompilerParams(
            dimension_semantics=("parallel",),
            vmem_limit_bytes=100 * 1024 * 1024),
    )(xt, poolb, w1p, b1p, w2p, b2p)
